# baseline (speedup 1.0000x reference)
import numpy as np
import jax
import jax.numpy as jnp
from jax.experimental import pallas as pl
from jax.experimental.pallas import tpu as pltpu

_C, _T, _H, _W = 3, 32, 256, 256
_ALPHA = 4
_NSLOW = _T // _ALPHA
_IDX = np.linspace(0.0, _T - 1, _NSLOW).astype(np.int32)
_FB = 8
_NCH = _C * _T // _FB
_SPB = _NSLOW * _FB // _T
_NBUF = 4

for _q in range(_T // _FB):
    for _j in range(_SPB):
        assert _FB * _q <= _IDX[_SPB * _q + _j] < _FB * (_q + 1)


def _body(in_hbm, slow_hbm, fast_hbm, *scratch):
    bufs = scratch[:_NBUF]
    rsems = scratch[_NBUF:2 * _NBUF]
    wsems = scratch[2 * _NBUF:3 * _NBUF]

    def chunk(k):
        c, q = divmod(k, _T // _FB)
        return c, q

    def read(k):
        c, q = chunk(k)
        b = k % _NBUF
        pltpu.make_async_copy(
            in_hbm.at[c, pl.ds(q * _FB, _FB)], bufs[b], rsems[b]).start()

    def write(k):
        c, q = chunk(k)
        b = k % _NBUF
        pltpu.make_async_copy(
            in_hbm.at[c, pl.ds(q * _FB, _FB)], bufs[b], rsems[b]).wait()
        pltpu.make_async_copy(
            bufs[b], fast_hbm.at[c, pl.ds(q * _FB, _FB)], wsems[b]).start()
        for j in range(_SPB):
            i = _SPB * q + j
            off = int(_IDX[i]) - _FB * q
            pltpu.make_async_copy(
                bufs[b].at[pl.ds(off, 1)], slow_hbm.at[c, pl.ds(i, 1)],
                wsems[b]).start()

    def wait_writes(k):
        c, q = chunk(k)
        b = k % _NBUF
        pltpu.make_async_copy(
            bufs[b], fast_hbm.at[c, pl.ds(q * _FB, _FB)], wsems[b]).wait()
        for j in range(_SPB):
            i = _SPB * q + j
            off = int(_IDX[i]) - _FB * q
            pltpu.make_async_copy(
                bufs[b].at[pl.ds(off, 1)], slow_hbm.at[c, pl.ds(i, 1)],
                wsems[b]).wait()

    for k in range(_NBUF):
        read(k)
    for k in range(_NCH):
        write(k)
        if k >= 1 and (k - 1) + _NBUF < _NCH:
            wait_writes(k - 1)
            read(k - 1 + _NBUF)
    for k in range(_NCH - _NBUF, _NCH):
        wait_writes(k)


def kernel(frames):
    slow, fast = pl.pallas_call(
        _body,
        in_specs=[pl.BlockSpec(memory_space=pl.ANY)],
        out_specs=[
            pl.BlockSpec(memory_space=pl.ANY),
            pl.BlockSpec(memory_space=pl.ANY),
        ],
        out_shape=[
            jax.ShapeDtypeStruct((_C, _NSLOW, _H, _W), jnp.float32),
            jax.ShapeDtypeStruct((_C, _T, _H, _W), jnp.float32),
        ],
        scratch_shapes=(
            [pltpu.VMEM((_FB, _H, _W), jnp.float32) for _ in range(_NBUF)]
            + [pltpu.SemaphoreType.DMA for _ in range(2 * _NBUF)]
        ),
    )(frames)
    return (slow, fast)


# speedup vs baseline: 1.1014x; 1.1014x over previous
import numpy as np
import jax
import jax.numpy as jnp
from jax.experimental import pallas as pl

_C, _T, _H, _W = 3, 32, 256, 256
_ALPHA = 4
_NSLOW = _T // _ALPHA
_IDX = np.linspace(0.0, _T - 1, _NSLOW).astype(np.int32)


def _body(in_ref, slow_ref, fast_ref):
    fast_ref[...] = in_ref[...]
    for s in range(_NSLOW):
        slow_ref[:, pl.ds(s, 1)] = in_ref[:, pl.ds(int(_IDX[s]), 1)]


def kernel(frames):
    slow, fast = pl.pallas_call(
        _body,
        grid=(_C,),
        in_specs=[pl.BlockSpec((1, _T, _H, _W), lambda c: (c, 0, 0, 0))],
        out_specs=[
            pl.BlockSpec((1, _NSLOW, _H, _W), lambda c: (c, 0, 0, 0)),
            pl.BlockSpec((1, _T, _H, _W), lambda c: (c, 0, 0, 0)),
        ],
        out_shape=[
            jax.ShapeDtypeStruct((_C, _NSLOW, _H, _W), jnp.float32),
            jax.ShapeDtypeStruct((_C, _T, _H, _W), jnp.float32),
        ],
    )(frames)
    return (slow, fast)
